# baseline (speedup 1.0000x reference)
import jax
import jax.numpy as jnp
from jax import lax
from jax.experimental import pallas as pl
from jax.experimental.pallas import tpu as pltpu
from jax.experimental.pallas import tpu_sc as plsc

_NUM_ROWS = 100001
_D = 64
_B = 16384

_INFO = plsc.get_sparse_core_info()
_NC = _INFO.num_cores
_NS = _INFO.num_subcores
_NW = _NC * _NS
_RPW = _D // _NW

_P = 33408
_P2 = 33152
_TAIL = _NUM_ROWS - 2 * _P - _P2
_OFFS = (0, _P, 2 * _P)
_SWEEP = (_P, _P, _P2 + _TAIL)


def _gather_body(labels_hbm, tableT_hbm, tailT_hbm, outT_hbm,
                 buf0, buf1, lab_v, out0, out1, shlab_v,
                 sem_lab, sem_row, sem_out):
    sid = lax.axis_index("s")
    wid = sid * _NC + lax.axis_index("c")
    bufs = (buf0, buf1)
    outs = (out0, out1)
    iota = lax.iota(jnp.int32, 16)

    def issue(gpc):
        r, k = divmod(gpc, 3)
        c = wid * _RPW + r
        buf = bufs[gpc % 2]
        if k < 2:
            return [pltpu.async_copy(
                tableT_hbm.at[c, pl.ds(_OFFS[k], _P)],
                buf.at[pl.ds(0, _P)], sem_row)]
        return [
            pltpu.async_copy(tableT_hbm.at[c, pl.ds(_OFFS[2], _P2)],
                             buf.at[pl.ds(0, _P2)], sem_row),
            pltpu.async_copy(tailT_hbm.at[c], buf.at[pl.ds(_P2, 128)],
                             sem_row),
        ]

    pending = issue(0)

    @pl.when(sid == 0)
    def _stage_labels():
        pltpu.async_copy(labels_hbm, shlab_v, sem_lab).wait()

    plsc.subcore_barrier()
    pltpu.async_copy(shlab_v, lab_v, sem_lab).wait()

    out_cps = []
    for r in range(_RPW):
        for k in range(3):
            gpc = r * 3 + k
            for cp in pending:
                cp.wait()
            pending = issue(gpc + 1) if gpc + 1 < _RPW * 3 else []
            rbuf = bufs[gpc % 2]
            oref = outs[r]
            lo = _OFFS[k]
            sz = _SWEEP[k]

            if k == 0:
                @plsc.parallel_loop(0, _B, step=16, unroll=4)
                def _sweep(i, _rbuf=rbuf, _oref=oref):
                    idx = lab_v[pl.ds(i, 16)]
                    relc = jnp.minimum(idx, _P - 1)
                    vals = plsc.load_gather(_rbuf, [relc])
                    plsc.store_scatter(_oref, [iota + i], vals)
            else:
                @plsc.parallel_loop(0, _B, step=16, unroll=4)
                def _sweep(i, _rbuf=rbuf, _oref=oref, _lo=lo, _sz=sz):
                    idx = lab_v[pl.ds(i, 16)]
                    rel = idx - _lo
                    m = plsc.bitcast(rel, jnp.uint32) < jnp.uint32(_sz)
                    vals = plsc.load_gather(_rbuf, [rel], mask=m)
                    plsc.store_scatter(_oref, [iota + i], vals, mask=m)

        out_cps.append(pltpu.async_copy(
            outs[r], outT_hbm.at[wid * _RPW + r], sem_out))
    for cp in out_cps:
        cp.wait()


def kernel(labels, table):
    mesh = plsc.VectorSubcoreMesh(core_axis_name="c", subcore_axis_name="s")
    gather = pl.kernel(
        _gather_body,
        out_type=jax.ShapeDtypeStruct((_D, _B), jnp.float32),
        mesh=mesh,
        scratch_types=[
            pltpu.VMEM((_P,), jnp.float32),
            pltpu.VMEM((_P,), jnp.float32),
            pltpu.VMEM((_B,), jnp.int32),
            pltpu.VMEM((_B,), jnp.float32),
            pltpu.VMEM((_B,), jnp.float32),
            pltpu.VMEM_SHARED((_B,), jnp.int32),
            pltpu.SemaphoreType.DMA,
            pltpu.SemaphoreType.DMA,
            pltpu.SemaphoreType.DMA,
        ],
        compiler_params=pltpu.CompilerParams(
            disable_bounds_checks=True,
            disable_semaphore_checks=True,
            skip_device_barrier=True,
            needs_layout_passes=False,
        ),
    )
    tableT = table.T
    tailT = lax.pad(tableT, jnp.float32(0),
                    ((0, 0, 0), (-(2 * _P + _P2), 128 - _TAIL, 0)))
    outT = gather(labels.astype(jnp.int32), tableT, tailT)
    return outT.T


# speedup vs baseline: 1.0013x; 1.0013x over previous
import jax
import jax.numpy as jnp
from jax import lax
from jax.experimental import pallas as pl
from jax.experimental.pallas import tpu as pltpu
from jax.experimental.pallas import tpu_sc as plsc

_NUM_ROWS = 100001
_D = 64
_B = 16384

_INFO = plsc.get_sparse_core_info()
_NC = _INFO.num_cores
_NS = _INFO.num_subcores
_NW = _NC * _NS
_RPW = _D // _NW

_P = 33408
_P2 = 33152
_TAIL = _NUM_ROWS - 2 * _P - _P2
_OFFS = (0, _P, 2 * _P)
_SWEEP = (_P, _P, _P2 + _TAIL)


def _gather_body(labels_hbm, tableT_hbm, tailT_hbm, outT_hbm,
                 buf0, buf1, lab_v, out0, out1, shlab_v,
                 sem_lab, sem_row, sem_out):
    sid = lax.axis_index("s")
    wid = sid * _NC + lax.axis_index("c")
    bufs = (buf0, buf1)
    outs = (out0, out1)
    iota = lax.iota(jnp.int32, 16)

    def issue(gpc):
        r, k = divmod(gpc, 3)
        c = wid * _RPW + r
        buf = bufs[gpc % 2]
        if k < 2:
            return [pltpu.async_copy(
                tableT_hbm.at[c, pl.ds(_OFFS[k], _P)],
                buf.at[pl.ds(0, _P)], sem_row)]
        return [
            pltpu.async_copy(tableT_hbm.at[c, pl.ds(_OFFS[2], _P2)],
                             buf.at[pl.ds(0, _P2)], sem_row),
            pltpu.async_copy(tailT_hbm.at[c], buf.at[pl.ds(_P2, 128)],
                             sem_row),
        ]

    pending = issue(0)

    @pl.when(sid == 0)
    def _stage_labels():
        pltpu.async_copy(labels_hbm, shlab_v, sem_lab).wait()

    plsc.subcore_barrier()
    pltpu.async_copy(shlab_v, lab_v, sem_lab).wait()

    out_cps = []
    for r in range(_RPW):
        for k in range(3):
            gpc = r * 3 + k
            for cp in pending:
                cp.wait()
            pending = issue(gpc + 1) if gpc + 1 < _RPW * 3 else []
            rbuf = bufs[gpc % 2]
            oref = outs[r]
            lo = _OFFS[k]
            sz = _SWEEP[k]

            if k == 0:
                @plsc.parallel_loop(0, _B, step=16, unroll=4)
                def _sweep(i, _rbuf=rbuf, _oref=oref):
                    idx = lab_v[pl.ds(i, 16)]
                    relc = jnp.minimum(idx, _P - 1)
                    vals = plsc.load_gather(_rbuf, [relc])
                    plsc.store_scatter(_oref, [iota + i], vals)
            else:
                @plsc.parallel_loop(0, _B, step=16, unroll=4)
                def _sweep(i, _rbuf=rbuf, _oref=oref, _lo=lo, _sz=sz):
                    idx = lab_v[pl.ds(i, 16)]
                    rel = idx - _lo
                    m = plsc.bitcast(rel, jnp.uint32) < jnp.uint32(_sz)
                    vals = plsc.load_gather(_rbuf, [rel], mask=m)
                    plsc.store_scatter(_oref, [iota + i], vals, mask=m)

        out_cps.append(pltpu.async_copy(
            outs[r], outT_hbm.at[wid * _RPW + r], sem_out))
    for cp in out_cps:
        cp.wait()


def kernel(labels, table):
    mesh = plsc.VectorSubcoreMesh(core_axis_name="c", subcore_axis_name="s")
    gather = pl.kernel(
        _gather_body,
        out_type=jax.ShapeDtypeStruct((_D, _B), jnp.float32),
        mesh=mesh,
        scratch_types=[
            pltpu.VMEM((_P,), jnp.float32),
            pltpu.VMEM((_P,), jnp.float32),
            pltpu.VMEM((_B,), jnp.int32),
            pltpu.VMEM((_B,), jnp.float32),
            pltpu.VMEM((_B,), jnp.float32),
            pltpu.VMEM_SHARED((_B,), jnp.int32),
            pltpu.SemaphoreType.DMA,
            pltpu.SemaphoreType.DMA,
            pltpu.SemaphoreType.DMA,
        ],
        compiler_params=pltpu.CompilerParams(
            disable_bounds_checks=True,
            disable_semaphore_checks=True,
            skip_device_barrier=True,
            needs_layout_passes=False,
        ),
    )
    tableT = table.T
    tailT = jnp.pad(
        lax.slice_in_dim(tableT, 2 * _P + _P2, _NUM_ROWS, axis=1),
        ((0, 0), (0, 128 - _TAIL)))
    outT = gather(labels.astype(jnp.int32), tableT, tailT)
    return outT.T
